# baseline (speedup 1.0000x reference)
import functools

import jax
import jax.numpy as jnp
from jax import lax
from jax.experimental import pallas as pl
from jax.experimental.pallas import tpu as pltpu
from jax.experimental.pallas import tpu_sc as plsc

NV = 10000
NP = 10240
NNZ = 320000
CHUNK = 128
NCHUNK = 80
PER_TILE = CHUNK * NCHUNK
NNZP = PER_TILE * 32
RPT = NP // 16
BLK = 256


def _sc_mesh():
    return plsc.VectorSubcoreMesh(core_axis_name="c", subcore_axis_name="s")


@functools.cache
def _accum_kernel(C, with_deg):
    nout = 4 if with_deg else 2
    out = ([jax.ShapeDtypeStruct((NP, C), jnp.float32) for _ in range(2)] +
           [jax.ShapeDtypeStruct((NP, 16), jnp.float32)
            for _ in range(nout - 2)])
    scratch = [
        pltpu.VMEM((NCHUNK, CHUNK), jnp.int32),
        pltpu.VMEM((NCHUNK, CHUNK), jnp.int32),
        pltpu.VMEM((NCHUNK, CHUNK), jnp.float32),
        pltpu.VMEM((CHUNK, C), jnp.float32),
        pltpu.VMEM((CHUNK, C), jnp.float32),
        pltpu.VMEM((CHUNK, C), jnp.float32),
        pltpu.VMEM((CHUNK, C), jnp.float32),
        pltpu.VMEM_SHARED((NP, C), jnp.float32),
    ] + [pltpu.SemaphoreType.DMA] * 8
    if with_deg:
        scratch += ([pltpu.VMEM((CHUNK, 16), jnp.float32)] * 4 +
                    [pltpu.VMEM_SHARED((NP, 16), jnp.float32)] +
                    [pltpu.SemaphoreType.DMA] * 4)

    @functools.partial(
        pl.kernel, out_type=out, mesh=_sc_mesh(), scratch_types=scratch,
        compiler_params=pltpu.CompilerParams(use_tc_tiling_on_sc=False))
    def k(table_h, src_h, dst_h, w_h, out0, out1, *rest):
        if with_deg:
            (dg0, dg1, srcb, dstb, wb, b0, b1, b2, b3, acc,
             gs0, gs1, gs2, gs3, ss0, ss1, ss2, ss3,
             d0, d1, d2, d3, dacc, ds0, ds1, ds2, ds3) = rest
            dbufs = (d0, d1, d2, d3)
            dss = (ds0, ds1, ds2, ds3)
        else:
            (srcb, dstb, wb, b0, b1, b2, b3, acc,
             gs0, gs1, gs2, gs3, ss0, ss1, ss2, ss3) = rest
        bufs = (b0, b1, b2, b3)
        gs = (gs0, gs1, gs2, gs3)
        ss = (ss0, ss1, ss2, ss3)
        cid = lax.axis_index("c")
        sid = lax.axis_index("s")
        pltpu.sync_copy(src_h.at[cid, sid], srcb)
        pltpu.sync_copy(dst_h.at[cid, sid], dstb)
        pltpu.sync_copy(w_h.at[cid, sid], wb)

        zero = jnp.zeros((16,), jnp.float32)

        @pl.loop(0, CHUNK)
        def _(r):
            for j in range(C // 16):
                b0[r, pl.ds(16 * j, 16)] = zero
        base = sid * RPT
        for kk in range(RPT // CHUNK):
            pltpu.sync_copy(b0, acc.at[pl.ds(base + kk * CHUNK, CHUNK)])
        if with_deg:
            @pl.loop(0, CHUNK)
            def _(r):
                d0[r, :] = zero
            for kk in range(RPT // CHUNK):
                pltpu.sync_copy(d0, dacc.at[pl.ds(base + kk * CHUNK, CHUNK)])
        plsc.subcore_barrier()

        def fire(g, buf, sem):
            pltpu.async_copy(table_h.at[srcb.at[g]], buf, sem)

        def drain(buf, sem):
            pltpu.make_async_copy(table_h.at[srcb.at[0]], buf, sem).wait()

        def drain_d(dbuf, sem):
            pltpu.make_async_copy(dg0.at[pl.ds(0, CHUNK)], dbuf, sem).wait()

        def scale(g, buf, dbuf):
            @pl.loop(0, CHUNK // 16)
            def _(kk):
                w_vec = wb[g, pl.ds(kk * 16, 16)]
                for i in range(16):
                    s = jnp.full((16,), w_vec[i], jnp.float32)
                    r = kk * 16 + i
                    if with_deg:
                        dbuf[r, :] = s
                    for j in range(C // 16):
                        buf[r, pl.ds(16 * j, 16)] = (
                            buf[r, pl.ds(16 * j, 16)] * s)

        fire(0, bufs[0], gs[0])
        fire(1, bufs[1], gs[1])

        @pl.loop(0, NCHUNK // 4)
        def _(h):
            for b in range(4):
                g = h * 4 + b
                bb = (b + 2) % 4

                @pl.when(g >= 2)
                def _():
                    drain(bufs[bb], ss[bb])
                    if with_deg:
                        drain_d(dbufs[bb], dss[bb])

                @pl.when(g + 2 < NCHUNK)
                def _():
                    fire(g + 2, bufs[bb], gs[bb])

                drain(bufs[b], gs[b])
                scale(g, bufs[b], dbufs[b] if with_deg else None)
                pltpu.async_copy(bufs[b], acc.at[dstb.at[g]], ss[b],
                                 add=True)
                if with_deg:
                    pltpu.async_copy(dbufs[b], dacc.at[dstb.at[g]],
                                     dss[b], add=True)

        for t in (NCHUNK - 2, NCHUNK - 1):
            drain(bufs[t % 4], ss[t % 4])
            if with_deg:
                drain_d(dbufs[t % 4], dss[t % 4])
        plsc.subcore_barrier()
        for kk in range(RPT // CHUNK):
            sl = pl.ds(base + kk * CHUNK, CHUNK)
            pltpu.sync_copy(acc.at[sl], b0)
            if with_deg:
                pltpu.sync_copy(dacc.at[sl], d0)

            @pl.when(cid == 0)
            def _():
                pltpu.sync_copy(b0, out0.at[sl])
                if with_deg:
                    pltpu.sync_copy(d0, dg0.at[sl])

            @pl.when(cid == 1)
            def _():
                pltpu.sync_copy(b0, out1.at[sl])
                if with_deg:
                    pltpu.sync_copy(d0, dg1.at[sl])

    return k


def _tc_mm(x, w, b):
    n, kdim = x.shape
    c = w.shape[1]

    def body(x_ref, w_ref, b_ref, o_ref):
        o_ref[...] = (jnp.dot(x_ref[...], w_ref[...],
                              preferred_element_type=jnp.float32) + b_ref[...])

    return pl.pallas_call(
        body,
        grid=(n // BLK,),
        in_specs=[pl.BlockSpec((BLK, kdim), lambda i: (i, 0)),
                  pl.BlockSpec((kdim, c), lambda i: (0, 0)),
                  pl.BlockSpec((1, c), lambda i: (0, 0))],
        out_specs=pl.BlockSpec((BLK, c), lambda i: (i, 0)),
        out_shape=jax.ShapeDtypeStruct((n, c), jnp.float32),
    )(x, w, b)


def _tc_comb(p0, p1, d0, d1):
    n, c = p0.shape

    def body(p0r, p1r, d0r, d1r, o_ref):
        d = d0r[...] + d1r[...]
        num = p0r[...] + p1r[...]
        pos = d > 0
        o_ref[...] = jnp.where(pos, num / jnp.where(pos, d, 1.0), 0.0)

    return pl.pallas_call(
        body,
        grid=(n // BLK,),
        in_specs=[pl.BlockSpec((BLK, c), lambda i: (i, 0)),
                  pl.BlockSpec((BLK, c), lambda i: (i, 0)),
                  pl.BlockSpec((BLK, 1), lambda i: (i, 0)),
                  pl.BlockSpec((BLK, 1), lambda i: (i, 0))],
        out_specs=pl.BlockSpec((BLK, c), lambda i: (i, 0)),
        out_shape=jax.ShapeDtypeStruct((n, c), jnp.float32),
    )(p0, p1, d0, d1)


def _tc_comb_relu(q0, q1, d0, d1):
    n, c = q0.shape

    def body(q0r, q1r, d0r, d1r, o_ref):
        d = d0r[...] + d1r[...]
        num = q0r[...] + q1r[...]
        pos = d > 0
        xv = jnp.where(pos, num / jnp.where(pos, d, 1.0), 0.0)
        o_ref[...] = jnp.maximum(xv, 0.0)

    return pl.pallas_call(
        body,
        grid=(n // BLK,),
        in_specs=[pl.BlockSpec((BLK, c), lambda i: (i, 0)),
                  pl.BlockSpec((BLK, c), lambda i: (i, 0)),
                  pl.BlockSpec((BLK, 1), lambda i: (i, 0)),
                  pl.BlockSpec((BLK, 1), lambda i: (i, 0))],
        out_specs=pl.BlockSpec((BLK, c), lambda i: (i, 0)),
        out_shape=jax.ShapeDtypeStruct((n, c), jnp.float32),
    )(q0, q1, d0, d1)


def _tc_comb_mm_mask(p0, p1, d0, d1, w, b):
    n, c = p0.shape
    c2 = w.shape[1]

    def body(p0r, p1r, d0r, d1r, wr, br, mo, yo):
        d = d0r[...] + d1r[...]
        num = p0r[...] + p1r[...]
        pos = d > 0
        m = jnp.where(pos, num / jnp.where(pos, d, 1.0), 0.0)
        mo[...] = m
        y = (jnp.dot(m, wr[...],
                     preferred_element_type=jnp.float32) + br[...])
        yo[...] = jnp.where(pos, y, 0.0)

    return pl.pallas_call(
        body,
        grid=(n // BLK,),
        in_specs=[pl.BlockSpec((BLK, c), lambda i: (i, 0)),
                  pl.BlockSpec((BLK, c), lambda i: (i, 0)),
                  pl.BlockSpec((BLK, 1), lambda i: (i, 0)),
                  pl.BlockSpec((BLK, 1), lambda i: (i, 0)),
                  pl.BlockSpec((c, c2), lambda i: (0, 0)),
                  pl.BlockSpec((1, c2), lambda i: (0, 0))],
        out_specs=[pl.BlockSpec((BLK, c), lambda i: (i, 0)),
                   pl.BlockSpec((BLK, c2), lambda i: (i, 0))],
        out_shape=[jax.ShapeDtypeStruct((n, c), jnp.float32),
                   jax.ShapeDtypeStruct((n, c2), jnp.float32)],
    )(p0, p1, d0, d1, w, b)


def kernel(X, hg, v2e_weight, e2v_weight, W1, b1, W2, b2):
    vid = hg[0]
    eid = hg[1]
    padn = NNZP - NNZ
    pad_idx = jnp.full((padn,), NP - 1, jnp.int32)
    pad_w = jnp.zeros((padn,), jnp.float32)
    shape4 = (2, 16, NCHUNK, CHUNK)
    vid_p = jnp.concatenate([vid, pad_idx]).reshape(shape4)
    eid_p = jnp.concatenate([eid, pad_idx]).reshape(shape4)
    wv_p = jnp.concatenate([v2e_weight, pad_w]).reshape(shape4)
    we_p = jnp.concatenate([e2v_weight, pad_w]).reshape(shape4)
    X_p = jnp.pad(X, ((0, NP - NV), (0, 0)))
    b1r = b1.reshape(1, -1)
    b2r = b2.reshape(1, -1)

    t1 = _tc_mm(X_p, W1, b1r)
    p0, p1, de0, de1 = _accum_kernel(64, True)(t1, vid_p, eid_p, wv_p)
    de0c, de1c = de0[:, :1], de1[:, :1]
    xe1 = _tc_comb(p0, p1, de0c, de1c)
    q0, q1, dv0, dv1 = _accum_kernel(64, True)(xe1, eid_p, vid_p, we_p)
    dv0c, dv1c = dv0[:, :1], dv1[:, :1]
    xn1 = _tc_comb_relu(q0, q1, dv0c, dv1c)
    r0, r1 = _accum_kernel(64, False)(xn1, vid_p, eid_p, wv_p)
    m1, xe2 = _tc_comb_mm_mask(r0, r1, de0c, de1c, W2, b2r)
    s0, s1 = _accum_kernel(64, False)(m1, eid_p, vid_p, we_p)
    _, xn2 = _tc_comb_mm_mask(s0, s1, dv0c, dv1c, W2, b2r)

    return (xn1[:NV], xe1[:NV], xn2[:NV], xe2[:NV])
